# baseline (device time: 31404 ns/iter reference)
import jax
import jax.numpy as jnp
from jax import lax
from jax.experimental import pallas as pl
from jax.experimental.pallas import tpu as pltpu

N_DEV = 4
G = 2
SUB = 128
RS = 4


def kernel(x):
    m, n = x.shape
    ng = n // G
    rb = m // RS

    def body(x_ref, out_ref, hold, carry, send_buf, gather_ref,
             send_sems, recv_sems):
        i = pl.program_id(0)
        my = lax.axis_index("i")

        a0 = i < RS
        mid = (i >= RS) & (i < 3 * RS)
        is_a1 = mid & (i % 2 == 0)
        is_b0 = mid & (i % 2 == 1)
        is_b1 = i >= 3 * RS

        is_a = a0 | is_a1
        gA = jnp.where(a0, 0, 1)
        rA = jnp.where(a0, i, (i - RS) // 2)
        is_b = is_b0 | is_b1
        gB = jnp.where(is_b1, 1, 0)
        rB = jnp.where(is_b1, i - 3 * RS, (i - RS - 1) // 2)

        @pl.when(i == 0)
        def _():
            bsem = pltpu.get_barrier_semaphore()
            for off in range(1, N_DEV):
                nbr = (my + off) % N_DEV
                pl.semaphore_signal(
                    bsem, inc=1, device_id=(nbr,),
                    device_id_type=pl.DeviceIdType.MESH,
                )
            pl.semaphore_wait(bsem, N_DEV - 1)
            carry[...] = jnp.zeros_like(carry)

        @pl.when((i == RS - 1) | (i == 3 * RS - 2))
        def _():
            g = jnp.where(i == RS - 1, 0, 1)
            send_buf[...] = carry[pl.ds(g * 8, 1), :] + jnp.sum(
                x_ref[...], axis=0, keepdims=True
            )
            for off in range(1, N_DEV):
                nbr = (my + off) % N_DEV
                pltpu.make_async_remote_copy(
                    src_ref=send_buf,
                    dst_ref=gather_ref.at[pl.ds(g * 8 + my, 1)],
                    send_sem=send_sems.at[g * (N_DEV - 1) + off - 1],
                    recv_sem=recv_sems.at[g * N_DEV + my],
                    device_id=(nbr,),
                    device_id_type=pl.DeviceIdType.MESH,
                ).start()

        @pl.when(is_a)
        def _():
            ri = lax.broadcasted_iota(jnp.int32, (SUB, SUB), 0)
            ci = lax.broadcasted_iota(jnp.int32, (SUB, SUB), 1)
            L = (ri >= ci).astype(jnp.bfloat16)
            cur = carry[pl.ds(gA * 8, 1), :]
            base = gA * m + rA * rb
            for s in range(rb // SUB):
                xs = x_ref[s * SUB:(s + 1) * SUB, :].astype(jnp.bfloat16)
                y = lax.dot_general(
                    L, xs, (((1,), (0,)), ((), ())),
                    preferred_element_type=jnp.float32,
                ) + cur
                hold[pl.ds(base + s * SUB, SUB), :] = y.astype(jnp.bfloat16)
                cur = y[SUB - 1:SUB, :]
            carry[pl.ds(gA * 8, 1), :] = cur

        @pl.when((i == RS + 1) | (i == 3 * RS))
        def _():
            g = jnp.where(i == RS + 1, 0, 1)
            for off in range(1, N_DEV):
                src = (my + off) % N_DEV
                pltpu.make_async_remote_copy(
                    src_ref=send_buf,
                    dst_ref=gather_ref.at[pl.ds(g * 8 + src, 1)],
                    send_sem=send_sems.at[g * (N_DEV - 1) + off - 1],
                    recv_sem=recv_sems.at[g * N_DEV + src],
                    device_id=(my,),
                    device_id_type=pl.DeviceIdType.MESH,
                ).wait_recv()
            for off in range(1, N_DEV):
                nbr = (my + off) % N_DEV
                pltpu.make_async_remote_copy(
                    src_ref=send_buf,
                    dst_ref=gather_ref.at[pl.ds(g * 8 + my, 1)],
                    send_sem=send_sems.at[g * (N_DEV - 1) + off - 1],
                    recv_sem=recv_sems.at[g * N_DEV + my],
                    device_id=(nbr,),
                    device_id_type=pl.DeviceIdType.MESH,
                ).wait_send()

        @pl.when(is_b)
        def _():
            gath = gather_ref[pl.ds(gB * 8, 8), :]
            row_ids = lax.broadcasted_iota(jnp.int32, (8, ng), 0)
            off_row = jnp.sum(
                jnp.where(row_ids < my, gath, 0.0), axis=0, keepdims=True
            ).astype(jnp.bfloat16)
            base = gB * m + rB * rb
            sr = rb // 4
            for s in range(4):
                out_ref[s * sr:(s + 1) * sr, :] = (
                    hold[pl.ds(base + s * sr, sr), :] + off_row
                )

    def in_map(i):
        a0 = i < RS
        mid = (i >= RS) & (i < 3 * RS)
        is_a1 = mid & (i % 2 == 0)
        row = jnp.where(
            a0, i,
            jnp.where(mid,
                      jnp.where(is_a1, (i - RS) // 2, (i - RS - 1) // 2),
                      RS - 1),
        )
        col = jnp.where(a0, 0, 1)
        return (row, col)

    def out_map(i):
        mid = (i >= RS) & (i < 3 * RS)
        is_b0 = mid & (i % 2 == 1)
        is_b1 = i >= 3 * RS
        row = jnp.where(
            is_b1, i - 3 * RS,
            jnp.where(is_b0, (i - RS - 1) // 2,
                      jnp.maximum((i - RS - 2) // 2, 0)),
        )
        col = jnp.where(is_b1, 1, 0)
        return (row, col)

    return pl.pallas_call(
        body,
        out_shape=jax.ShapeDtypeStruct((m, n), jnp.bfloat16),
        grid=(4 * RS,),
        in_specs=[
            pl.BlockSpec((rb, ng), in_map, memory_space=pltpu.VMEM)
        ],
        out_specs=pl.BlockSpec((rb, ng), out_map, memory_space=pltpu.VMEM),
        scratch_shapes=[
            pltpu.VMEM((G * m, ng), jnp.bfloat16),
            pltpu.VMEM((G * 8, ng), jnp.float32),
            pltpu.VMEM((1, ng), jnp.float32),
            pltpu.VMEM((G * 8, ng), jnp.float32),
            pltpu.SemaphoreType.DMA((G * (N_DEV - 1),)),
            pltpu.SemaphoreType.DMA((G * N_DEV,)),
        ],
        compiler_params=pltpu.CompilerParams(collective_id=0),
    )(x)


# device time: 25628 ns/iter; 1.2254x vs baseline; 1.2254x over previous
import jax
import jax.numpy as jnp
from jax import lax
from jax.experimental import pallas as pl
from jax.experimental.pallas import tpu as pltpu

N_DEV = 4
B = 1024
OB = 2048
SUB = 128


def _cumsum_tiles(x_ref, cur, store, ns):
    ri = lax.broadcasted_iota(jnp.int32, (SUB, SUB), 0)
    ci = lax.broadcasted_iota(jnp.int32, (SUB, SUB), 1)
    L = (ri >= ci).astype(jnp.bfloat16)
    for s in range(ns):
        xs = x_ref[s * SUB:(s + 1) * SUB, :].astype(jnp.bfloat16)
        y = lax.dot_general(
            L, xs, (((1,), (0,)), ((), ())),
            preferred_element_type=jnp.float32,
        ) + cur
        store(s, y)
        cur = y[SUB - 1:SUB, :]
    return cur


def kernel(x):
    m, n = x.shape
    nb = m // B
    nob = m // OB

    def body(x_ref, out_ref, hold, carry, offs, send_buf, gather_ref,
             send_sems, recv_sems):
        step = pl.program_id(0)
        my = lax.axis_index("i")

        @pl.when(step == 0)
        def _():
            bsem = pltpu.get_barrier_semaphore()
            for off in range(1, N_DEV):
                nbr = (my + off) % N_DEV
                pl.semaphore_signal(
                    bsem, inc=1, device_id=(nbr,),
                    device_id_type=pl.DeviceIdType.MESH,
                )
            pl.semaphore_wait(bsem, N_DEV - 1)
            carry[...] = jnp.zeros_like(carry)

        @pl.when(step < nb - 1)
        def _():
            def store(s, y):
                hold[pl.ds(step * B + s * SUB, SUB), :] = (
                    y.astype(jnp.bfloat16)
                )
            carry[...] = _cumsum_tiles(x_ref, carry[...], store, B // SUB)

        @pl.when(step == nb - 1)
        def _():
            send_buf[...] = carry[...] + jnp.sum(
                x_ref[...], axis=0, keepdims=True
            )
            for off in range(1, N_DEV):
                nbr = (my + off) % N_DEV
                pltpu.make_async_remote_copy(
                    src_ref=send_buf,
                    dst_ref=gather_ref.at[pl.ds(my, 1)],
                    send_sem=send_sems.at[off - 1],
                    recv_sem=recv_sems.at[my],
                    device_id=(nbr,),
                    device_id_type=pl.DeviceIdType.MESH,
                ).start()

        @pl.when(step == nb)
        def _():
            for off in range(1, N_DEV):
                src = (my + off) % N_DEV
                pltpu.make_async_remote_copy(
                    src_ref=send_buf,
                    dst_ref=gather_ref.at[pl.ds(src, 1)],
                    send_sem=send_sems.at[off - 1],
                    recv_sem=recv_sems.at[src],
                    device_id=(my,),
                    device_id_type=pl.DeviceIdType.MESH,
                ).wait_recv()
            for off in range(1, N_DEV):
                nbr = (my + off) % N_DEV
                pltpu.make_async_remote_copy(
                    src_ref=send_buf,
                    dst_ref=gather_ref.at[pl.ds(my, 1)],
                    send_sem=send_sems.at[off - 1],
                    recv_sem=recv_sems.at[my],
                    device_id=(nbr,),
                    device_id_type=pl.DeviceIdType.MESH,
                ).wait_send()
            row_ids = lax.broadcasted_iota(jnp.int32, (N_DEV, n), 0)
            offs[...] = jnp.sum(
                jnp.where(row_ids < my, gather_ref[...], 0.0),
                axis=0, keepdims=True,
            ).astype(jnp.bfloat16)

        @pl.when((step >= nb) & (step < nb + nob - 1))
        def _():
            base = (step - nb) * OB
            for h in range(OB // B):
                out_ref[h * B:(h + 1) * B, :] = (
                    hold[pl.ds(base + h * B, B), :] + offs[...]
                )

        @pl.when(step == nb + nob - 1)
        def _():
            base = (nob - 1) * OB
            out_ref[0:B, :] = hold[pl.ds(base, B), :] + offs[...]

            def store(s, y):
                out_ref[B + s * SUB:B + (s + 1) * SUB, :] = (
                    y.astype(jnp.bfloat16) + offs[...]
                )
            _cumsum_tiles(x_ref, carry[...], store, B // SUB)

    return pl.pallas_call(
        body,
        out_shape=jax.ShapeDtypeStruct((m, n), jnp.bfloat16),
        grid=(nb + nob,),
        in_specs=[
            pl.BlockSpec(
                (B, n), lambda b: (jnp.minimum(b, nb - 1), 0),
                memory_space=pltpu.VMEM,
            )
        ],
        out_specs=pl.BlockSpec(
            (OB, n), lambda b: (jnp.maximum(b - nb, 0), 0),
            memory_space=pltpu.VMEM,
        ),
        scratch_shapes=[
            pltpu.VMEM((m - B, n), jnp.bfloat16),
            pltpu.VMEM((1, n), jnp.float32),
            pltpu.VMEM((1, n), jnp.bfloat16),
            pltpu.VMEM((1, n), jnp.float32),
            pltpu.VMEM((N_DEV, n), jnp.float32),
            pltpu.SemaphoreType.DMA((N_DEV - 1,)),
            pltpu.SemaphoreType.DMA((N_DEV,)),
        ],
        compiler_params=pltpu.CompilerParams(collective_id=0),
    )(x)


# device time: 25216 ns/iter; 1.2454x vs baseline; 1.0163x over previous
import jax
import jax.numpy as jnp
from jax import lax
from jax.experimental import pallas as pl
from jax.experimental.pallas import tpu as pltpu

N_DEV = 4
B = 1024
SUB = 128


def kernel(x):
    m, n = x.shape
    nb = m // B

    def body(x_ref, out_ref, hold, carry, offs, send_buf, gather_ref,
             send_sems, recv_sems):
        step = pl.program_id(0)
        my = lax.axis_index("i")

        @pl.when(step == 0)
        def _():
            bsem = pltpu.get_barrier_semaphore()
            for off in range(1, N_DEV):
                nbr = (my + off) % N_DEV
                pl.semaphore_signal(
                    bsem, inc=1, device_id=(nbr,),
                    device_id_type=pl.DeviceIdType.MESH,
                )
            pl.semaphore_wait(bsem, N_DEV - 1)
            carry[...] = jnp.zeros_like(carry)

        @pl.when(step == nb - 1)
        def _():
            send_buf[...] = carry[...] + jnp.sum(
                x_ref[...], axis=0, keepdims=True
            )
            for off in range(1, N_DEV):
                nbr = (my + off) % N_DEV
                pltpu.make_async_remote_copy(
                    src_ref=send_buf,
                    dst_ref=gather_ref.at[pl.ds(my, 1)],
                    send_sem=send_sems.at[off - 1],
                    recv_sem=recv_sems.at[my],
                    device_id=(nbr,),
                    device_id_type=pl.DeviceIdType.MESH,
                ).start()

        @pl.when(step < nb)
        def _():
            x16 = x_ref[...].astype(jnp.bfloat16)
            r = lax.broadcasted_iota(jnp.int32, (SUB, SUB), 0)
            c = lax.broadcasted_iota(jnp.int32, (SUB, SUB), 1)
            L = (r >= c).astype(jnp.bfloat16)
            cur = carry[...]
            for s in range(B // SUB):
                y = lax.dot_general(
                    L, x16[s * SUB:(s + 1) * SUB, :],
                    (((1,), (0,)), ((), ())),
                    preferred_element_type=jnp.float32,
                ) + cur
                hold[pl.ds(step * B + s * SUB, SUB), :] = (
                    y.astype(jnp.bfloat16)
                )
                cur = y[SUB - 1:SUB, :]
            carry[...] = cur

        @pl.when(step == nb)
        def _():
            for off in range(1, N_DEV):
                src = (my + off) % N_DEV
                pltpu.make_async_remote_copy(
                    src_ref=send_buf,
                    dst_ref=gather_ref.at[pl.ds(src, 1)],
                    send_sem=send_sems.at[off - 1],
                    recv_sem=recv_sems.at[src],
                    device_id=(my,),
                    device_id_type=pl.DeviceIdType.MESH,
                ).wait_recv()
            for off in range(1, N_DEV):
                nbr = (my + off) % N_DEV
                pltpu.make_async_remote_copy(
                    src_ref=send_buf,
                    dst_ref=gather_ref.at[pl.ds(my, 1)],
                    send_sem=send_sems.at[off - 1],
                    recv_sem=recv_sems.at[my],
                    device_id=(nbr,),
                    device_id_type=pl.DeviceIdType.MESH,
                ).wait_send()
            row_ids = lax.broadcasted_iota(jnp.int32, (N_DEV, n), 0)
            offs[...] = jnp.sum(
                jnp.where(row_ids < my, gather_ref[...], 0.0),
                axis=0, keepdims=True,
            ).astype(jnp.bfloat16)

        @pl.when(step >= nb)
        def _():
            out_ref[...] = hold[pl.ds((step - nb) * B, B), :] + offs[...]

    return pl.pallas_call(
        body,
        out_shape=jax.ShapeDtypeStruct((m, n), jnp.bfloat16),
        grid=(2 * nb,),
        in_specs=[
            pl.BlockSpec(
                (B, n), lambda b: (jnp.minimum(b, nb - 1), 0),
                memory_space=pltpu.VMEM,
            )
        ],
        out_specs=pl.BlockSpec(
            (B, n), lambda b: (jnp.maximum(b - nb, 0), 0),
            memory_space=pltpu.VMEM,
        ),
        scratch_shapes=[
            pltpu.VMEM((m, n), jnp.bfloat16),
            pltpu.VMEM((1, n), jnp.float32),
            pltpu.VMEM((1, n), jnp.bfloat16),
            pltpu.VMEM((1, n), jnp.float32),
            pltpu.VMEM((N_DEV, n), jnp.float32),
            pltpu.SemaphoreType.DMA((N_DEV - 1,)),
            pltpu.SemaphoreType.DMA((N_DEV,)),
        ],
        compiler_params=pltpu.CompilerParams(collective_id=0),
    )(x)
